# baseline (speedup 1.0000x reference)
import functools

import jax
import jax.numpy as jnp
from jax import lax
from jax.experimental import pallas as pl
from jax.experimental.pallas import tpu as pltpu
from jax.experimental.pallas import tpu_sc as plsc


def kernel(user_id, timestamp, user_table, time_table, buckets):
    B = user_id.shape[0]
    UD = user_table.shape[1]
    TD = time_table.shape[1]
    NB = buckets.shape[0]

    info = plsc.get_sparse_core_info()
    NC, NS, L = info.num_cores, info.num_subcores, info.num_lanes
    NW = NC * NS
    bpw = B // NW
    nq = 4
    qrows = bpw // nq
    qch = qrows // (2 * L)

    mesh = plsc.VectorSubcoreMesh(core_axis_name="c", subcore_axis_name="s")

    @functools.partial(
        pl.kernel,
        out_type=jax.ShapeDtypeStruct((2 * B, UD), jnp.float32),
        mesh=mesh,
        compiler_params=pltpu.CompilerParams(
            needs_layout_passes=False, use_tc_tiling_on_sc=False
        ),
        scratch_types=[
            pltpu.VMEM((bpw,), jnp.int32),
            pltpu.VMEM((bpw,), jnp.float32),
            pltpu.VMEM((NB,), jnp.float32),
            pltpu.VMEM((bpw,), jnp.int32),
            pltpu.VMEM((bpw,), jnp.int32),
            pltpu.VMEM((bpw,), jnp.int32),
            pltpu.VMEM((2 * bpw, UD), jnp.float32),
            pltpu.SemaphoreType.DMA,
            pltpu.SemaphoreType.DMA,
            pltpu.SemaphoreType.DMA,
        ],
    )
    def body(uid_hbm, ts_hbm, utab_hbm, ttab_hbm, bkt_hbm, out_hbm,
             uidx_v, ts_v, bkt_v, tidx_v, srowu_v, srowt_v, rows_v, sem_u,
             sem_t, sem_o):
        wid = lax.axis_index("s") * NC + lax.axis_index("c")
        base = wid * bpw
        iota = lax.iota(jnp.int32, L)

        pltpu.sync_copy(uid_hbm.at[pl.ds(base, bpw)], uidx_v)
        ucopy = pltpu.async_copy(utab_hbm.at[uidx_v], rows_v.at[pl.ds(0, bpw)],
                                 sem_u)

        pltpu.sync_copy(bkt_hbm, bkt_v)
        pltpu.sync_copy(ts_hbm.at[pl.ds(base, bpw)], ts_v)

        def search_pair(c, carry):
            v0 = ts_v[pl.ds(2 * c * L, L)]
            v1 = ts_v[pl.ds((2 * c + 1) * L, L)]
            a0 = jnp.zeros((L,), jnp.int32)
            a1 = jnp.zeros((L,), jnp.int32)
            k = NB
            while k >= 1:
                p0 = jnp.minimum(a0 + (k - 1), NB - 1)
                p1 = jnp.minimum(a1 + (k - 1), NB - 1)
                b0 = plsc.load_gather(bkt_v, [p0])
                b1 = plsc.load_gather(bkt_v, [p1])
                a0 = jnp.where((b0 <= v0) & (a0 + k <= NB), a0 + k, a0)
                a1 = jnp.where((b1 <= v1) & (a1 + k <= NB), a1 + k, a1)
                k //= 2
            tidx_v[pl.ds(2 * c * L, L)] = a0
            tidx_v[pl.ds((2 * c + 1) * L, L)] = a1
            s = (base + 2 * c * L) * 2 + iota * 2
            srowu_v[pl.ds(2 * c * L, L)] = s
            srowu_v[pl.ds((2 * c + 1) * L, L)] = s + 2 * L
            srowt_v[pl.ds(2 * c * L, L)] = s + 1
            srowt_v[pl.ds((2 * c + 1) * L, L)] = s + 2 * L + 1
            return carry

        tq = []
        for q in range(nq):
            lax.fori_loop(q * qch, (q + 1) * qch, search_pair, 0)
            tq.append(pltpu.async_copy(
                ttab_hbm.at[tidx_v.at[pl.ds(q * qrows, qrows)]],
                rows_v.at[pl.ds(bpw + q * qrows, qrows)], sem_t))

        ucopy.wait()
        oc_u = pltpu.async_copy(rows_v.at[pl.ds(0, bpw)],
                                out_hbm.at[srowu_v], sem_o)
        for cp in tq:
            cp.wait()
        oc_t = pltpu.async_copy(rows_v.at[pl.ds(bpw, bpw)],
                                out_hbm.at[srowt_v], sem_o)
        oc_u.wait()
        oc_t.wait()

    out2 = body(user_id, timestamp, user_table, time_table, buckets)
    return out2.reshape(B, UD + TD)
